# Initial kernel scaffold; baseline (speedup 1.0000x reference)
import functools

import jax
import jax.numpy as jnp
from jax import lax
from jax.experimental import pallas as pl
from jax.experimental.pallas import tpu as pltpu
from jax.experimental.pallas import tpu_sc as plsc

N = 10000
E = 320000
DIN = 128
DH = 64
DZ = 32

NC = 2
NS = 16
NW = NC * NS
C = 128
K = 80
EW = K * C
E_PAD = NW * EW

N_PAD = 10240
NSLICE = N_PAD // NS

_MESH = dict(core_axis_name="c", subcore_axis_name="s", num_cores=NC,
             num_subcores=NS)


def _wids():
    cid = lax.axis_index("c")
    sid = lax.axis_index("s")
    return cid, sid, sid * NC + cid


def _deg_body(dstp, out, idx_v, ones_v, zb_v, shared):
    cid, sid, wid = _wids()
    for t in range(C // 16):
        ones_v[pl.ds(t * 16, 16)] = jnp.ones((16,), jnp.float32)
        zb_v[pl.ds(t * 16, 16)] = jnp.zeros((16,), jnp.float32)
    base = sid * NSLICE
    for t in range(NSLICE // C):
        pltpu.sync_copy(zb_v, shared.at[pl.ds(base + t * C, C)])
    pltpu.sync_copy(dstp.at[wid], idx_v)
    plsc.subcore_barrier()

    def step(j, carry):
        pltpu.sync_copy(ones_v, shared.at[idx_v.at[j]], add=True)
        return carry

    lax.fori_loop(0, K, step, 0)
    plsc.subcore_barrier()
    pltpu.sync_copy(shared.at[pl.ds(base, NSLICE)],
                    out.at[cid].at[pl.ds(base, NSLICE)])


_deg_call = functools.partial(
    pl.kernel,
    out_type=jax.ShapeDtypeStruct((NC, N_PAD), jnp.float32),
    mesh=plsc.VectorSubcoreMesh(**_MESH),
    scratch_types=[
        pltpu.VMEM((K, C), jnp.int32),
        pltpu.VMEM((C,), jnp.float32),
        pltpu.VMEM((C,), jnp.float32),
        pltpu.VMEM_SHARED((N_PAD,), jnp.float32),
    ],
)(_deg_body)


def _seg_body(table, srcp, dstp, out, idx_s, idx_d, rows, shared, gsem, *,
              depth):
    cid, sid, wid = _wids()

    def zrow(r, carry):
        for t in range(depth // 16):
            rows[r, pl.ds(t * 16, 16)] = jnp.zeros((16,), jnp.float32)
        return carry

    lax.fori_loop(0, C, zrow, 0)
    base = sid * NSLICE
    for t in range(NSLICE // C):
        pltpu.sync_copy(rows, shared.at[pl.ds(base + t * C, C)])
    pltpu.sync_copy(srcp.at[wid], idx_s)
    pltpu.sync_copy(dstp.at[wid], idx_d)
    plsc.subcore_barrier()

    def step(j, carry):
        pltpu.async_copy(table.at[idx_s.at[j]], rows, gsem).wait()
        pltpu.sync_copy(rows, shared.at[idx_d.at[j]], add=True)
        return carry

    lax.fori_loop(0, K, step, 0)
    plsc.subcore_barrier()
    for t in range(NSLICE // C):
        sl = pl.ds(base + t * C, C)
        pltpu.sync_copy(shared.at[sl], out.at[cid].at[sl])


def _make_seg_call(depth):
    return functools.partial(
        pl.kernel,
        out_type=jax.ShapeDtypeStruct((NC, N_PAD, depth), jnp.float32),
        mesh=plsc.VectorSubcoreMesh(**_MESH),
        scratch_types=[
            pltpu.VMEM((K, C), jnp.int32),
            pltpu.VMEM((K, C), jnp.int32),
            pltpu.VMEM((C, depth), jnp.float32),
            pltpu.VMEM_SHARED((N_PAD, depth), jnp.float32),
            pltpu.SemaphoreType.DMA,
        ],
    )(functools.partial(_seg_body, depth=depth))


_seg_call_h = _make_seg_call(DH)
_seg_call_z = _make_seg_call(DZ)


def _dec_body(ztab, srcp, dstp, out, idx_s, idx_d, zs, zd, fb, obuf, gsem):
    cid, sid, wid = _wids()
    pltpu.sync_copy(srcp.at[wid], idx_s)
    pltpu.sync_copy(dstp.at[wid], idx_d)
    iota = lax.iota(jnp.int32, 16)

    def chunk(j, carry):
        pltpu.async_copy(ztab.at[idx_s.at[j]], zs, gsem).wait()
        pltpu.async_copy(ztab.at[idx_d.at[j]], zd, gsem).wait()

        def group(g, carry2):
            def row(r, carry3):
                e = g * 16 + r
                v = (zs[e, pl.ds(0, 16)] * zd[e, pl.ds(0, 16)]
                     + zs[e, pl.ds(16, 16)] * zd[e, pl.ds(16, 16)])
                fb[r] = v
                return carry3

            lax.fori_loop(0, 16, row, 0)

            def col(cc, acc):
                cv = plsc.load_gather(
                    fb, [iota, jnp.full((16,), cc, jnp.int32)])
                return acc + cv

            acc = lax.fori_loop(0, 16, col, jnp.zeros((16,), jnp.float32))
            obuf[j, pl.ds(g * 16, 16)] = acc
            return carry2

        lax.fori_loop(0, C // 16, group, 0)
        return carry

    lax.fori_loop(0, K, chunk, 0)
    pltpu.sync_copy(obuf, out.at[wid])


_dec_call = functools.partial(
    pl.kernel,
    out_type=jax.ShapeDtypeStruct((NW, K, C), jnp.float32),
    mesh=plsc.VectorSubcoreMesh(**_MESH),
    scratch_types=[
        pltpu.VMEM((K, C), jnp.int32),
        pltpu.VMEM((K, C), jnp.int32),
        pltpu.VMEM((C, DZ), jnp.float32),
        pltpu.VMEM((C, DZ), jnp.float32),
        pltpu.VMEM((16, 16), jnp.float32),
        pltpu.VMEM((K, C), jnp.float32),
        pltpu.SemaphoreType.DMA,
    ],
)(_dec_body)


_B = 512
_GRID = N_PAD // _B


def _dinv_of(degt):
    return lax.rsqrt(degt[:, 0:1] + degt[:, 1:2] + 1.0)


def _tc1_body(degt_ref, x_ref, w1_ref, o_ref):
    dinv = _dinv_of(degt_ref[...])
    h = jnp.dot(x_ref[...], w1_ref[...], preferred_element_type=jnp.float32)
    o_ref[...] = h * dinv


def _tc1(degt, x_pad, w1):
    return pl.pallas_call(
        _tc1_body,
        grid=(_GRID,),
        in_specs=[
            pl.BlockSpec((_B, NC), lambda i: (i, 0)),
            pl.BlockSpec((_B, DIN), lambda i: (i, 0)),
            pl.BlockSpec((DIN, DH), lambda i: (0, 0)),
        ],
        out_specs=pl.BlockSpec((_B, DH), lambda i: (i, 0)),
        out_shape=jax.ShapeDtypeStruct((N_PAD, DH), jnp.float32),
    )(degt, x_pad, w1)


def _tc2_body(degt_ref, p_ref, hs1_ref, b1_ref, w2_ref, o_ref):
    dinv = _dinv_of(degt_ref[...])
    agg = p_ref[0] + p_ref[1] + hs1_ref[...]
    a1 = jnp.maximum(dinv * agg + b1_ref[...], 0.0)
    h2 = jnp.dot(a1, w2_ref[...], preferred_element_type=jnp.float32)
    o_ref[...] = h2 * dinv


def _tc2(degt, parts1, hs1, b1, w2):
    return pl.pallas_call(
        _tc2_body,
        grid=(_GRID,),
        in_specs=[
            pl.BlockSpec((_B, NC), lambda i: (i, 0)),
            pl.BlockSpec((NC, _B, DH), lambda i: (0, i, 0)),
            pl.BlockSpec((_B, DH), lambda i: (i, 0)),
            pl.BlockSpec((1, DH), lambda i: (0, 0)),
            pl.BlockSpec((DH, DZ), lambda i: (0, 0)),
        ],
        out_specs=pl.BlockSpec((_B, DZ), lambda i: (i, 0)),
        out_shape=jax.ShapeDtypeStruct((N_PAD, DZ), jnp.float32),
    )(degt, parts1, hs1, b1, w2)


def _tc3_body(degt_ref, p_ref, hs2_ref, b2_ref, o_ref):
    dinv = _dinv_of(degt_ref[...])
    agg = p_ref[0] + p_ref[1] + hs2_ref[...]
    o_ref[...] = dinv * agg + b2_ref[...]


def _tc3(degt, parts2, hs2, b2):
    return pl.pallas_call(
        _tc3_body,
        grid=(_GRID,),
        in_specs=[
            pl.BlockSpec((_B, NC), lambda i: (i, 0)),
            pl.BlockSpec((NC, _B, DZ), lambda i: (0, i, 0)),
            pl.BlockSpec((_B, DZ), lambda i: (i, 0)),
            pl.BlockSpec((1, DZ), lambda i: (0, 0)),
        ],
        out_specs=pl.BlockSpec((_B, DZ), lambda i: (i, 0)),
        out_shape=jax.ShapeDtypeStruct((N_PAD, DZ), jnp.float32),
    )(degt, parts2, hs2, b2)


@jax.jit
def kernel(x, edge_index, W1, b1, W2, b2):
    src = edge_index[0]
    dst = edge_index[1]
    pad = jnp.full((E_PAD - E,), N, jnp.int32)
    srcp = jnp.concatenate([src, pad]).reshape(NW, K, C)
    dstp = jnp.concatenate([dst, pad]).reshape(NW, K, C)
    x_pad = jnp.pad(x, ((0, N_PAD - N), (0, 0)))

    deg_parts = _deg_call(dstp)
    degt = deg_parts.T

    hs1 = _tc1(degt, x_pad, W1)
    parts1 = _seg_call_h(hs1, srcp, dstp)
    hs2 = _tc2(degt, parts1, hs1, b1.reshape(1, DH), W2)
    parts2 = _seg_call_z(hs2, srcp, dstp)
    z = _tc3(degt, parts2, hs2, b2.reshape(1, DZ))

    recon = _dec_call(z, srcp, dstp).reshape(-1)[:E]
    return z[:N], recon


# speedup vs baseline: 10.5433x; 10.5433x over previous
import functools

import jax
import jax.numpy as jnp
from jax import lax
from jax.experimental import pallas as pl
from jax.experimental.pallas import tpu as pltpu
from jax.experimental.pallas import tpu_sc as plsc

N = 10000
E = 320000
DIN = 128
DH = 64
DZ = 32

NC = 2
NS = 16
NW = NC * NS
C = 128
K = 80
EW = K * C
E_PAD = NW * EW

N_PAD = 10240
NSLICE = N_PAD // NS

_MESH = dict(core_axis_name="c", subcore_axis_name="s", num_cores=NC,
             num_subcores=NS)
_SC_PARAMS = pltpu.CompilerParams(use_tc_tiling_on_sc=False,
                                  needs_layout_passes=False)


def _wids():
    cid = lax.axis_index("c")
    sid = lax.axis_index("s")
    return cid, sid, sid * NC + cid


def _deg_body(dstp, out, idx_v, ones_v, zb_v, shared):
    cid, sid, wid = _wids()
    for t in range(C // 16):
        ones_v[pl.ds(t * 16, 16)] = jnp.ones((16,), jnp.float32)
        zb_v[pl.ds(t * 16, 16)] = jnp.zeros((16,), jnp.float32)
    base = sid * NSLICE
    for t in range(NSLICE // C):
        pltpu.sync_copy(zb_v, shared.at[pl.ds(base + t * C, C)])
    pltpu.sync_copy(dstp.at[wid], idx_v)
    plsc.subcore_barrier()

    def step(j, carry):
        pltpu.sync_copy(ones_v, shared.at[idx_v.at[j]], add=True)
        return carry

    lax.fori_loop(0, K, step, 0)
    plsc.subcore_barrier()
    pltpu.sync_copy(shared.at[pl.ds(base, NSLICE)],
                    out.at[cid].at[pl.ds(base, NSLICE)])


_deg_call = functools.partial(
    pl.kernel,
    out_type=jax.ShapeDtypeStruct((NC, N_PAD), jnp.float32),
    mesh=plsc.VectorSubcoreMesh(**_MESH),
    compiler_params=_SC_PARAMS,
    scratch_types=[
        pltpu.VMEM((K, C), jnp.int32),
        pltpu.VMEM((C,), jnp.float32),
        pltpu.VMEM((C,), jnp.float32),
        pltpu.VMEM_SHARED((N_PAD,), jnp.float32),
    ],
)(_deg_body)


def _seg_body(table, srcp, dstp, out, idx_s, idx_d, rows, shared, gsem, *,
              depth):
    cid, sid, wid = _wids()

    def zrow(r, carry):
        for t in range(depth // 16):
            rows[r, pl.ds(t * 16, 16)] = jnp.zeros((16,), jnp.float32)
        return carry

    lax.fori_loop(0, C, zrow, 0)
    base = sid * NSLICE
    for t in range(NSLICE // C):
        pltpu.sync_copy(rows, shared.at[pl.ds(base + t * C, C)])
    pltpu.sync_copy(srcp.at[wid], idx_s)
    pltpu.sync_copy(dstp.at[wid], idx_d)
    plsc.subcore_barrier()

    def step(j, carry):
        pltpu.async_copy(table.at[idx_s.at[j]], rows, gsem).wait()
        pltpu.sync_copy(rows, shared.at[idx_d.at[j]], add=True)
        return carry

    lax.fori_loop(0, K, step, 0)
    plsc.subcore_barrier()
    for t in range(NSLICE // C):
        sl = pl.ds(base + t * C, C)
        pltpu.sync_copy(shared.at[sl], out.at[cid].at[sl])


def _make_seg_call(depth):
    return functools.partial(
        pl.kernel,
        out_type=jax.ShapeDtypeStruct((NC, N_PAD, depth), jnp.float32),
        mesh=plsc.VectorSubcoreMesh(**_MESH),
        compiler_params=_SC_PARAMS,
        scratch_types=[
            pltpu.VMEM((K, C), jnp.int32),
            pltpu.VMEM((K, C), jnp.int32),
            pltpu.VMEM((C, depth), jnp.float32),
            pltpu.VMEM_SHARED((N_PAD, depth), jnp.float32),
            pltpu.SemaphoreType.DMA,
        ],
    )(functools.partial(_seg_body, depth=depth))


_seg_call_h = _make_seg_call(DH)
_seg_call_z = _make_seg_call(DZ)


def _dec_body(ztab, srcp, dstp, out, idx_s, idx_d, zs, zd, fb, obuf, gsem):
    cid, sid, wid = _wids()
    pltpu.sync_copy(srcp.at[wid], idx_s)
    pltpu.sync_copy(dstp.at[wid], idx_d)
    iota = lax.iota(jnp.int32, 16)

    def chunk(j, carry):
        pltpu.async_copy(ztab.at[idx_s.at[j]], zs, gsem).wait()
        pltpu.async_copy(ztab.at[idx_d.at[j]], zd, gsem).wait()

        def group(g, carry2):
            def row(r, carry3):
                e = g * 16 + r
                v = (zs[e, pl.ds(0, 16)] * zd[e, pl.ds(0, 16)]
                     + zs[e, pl.ds(16, 16)] * zd[e, pl.ds(16, 16)])
                fb[r] = v
                return carry3

            lax.fori_loop(0, 16, row, 0)

            def col(cc, acc):
                cv = plsc.load_gather(
                    fb, [iota, jnp.full((16,), cc, jnp.int32)])
                return acc + cv

            acc = lax.fori_loop(0, 16, col, jnp.zeros((16,), jnp.float32))
            obuf[j, pl.ds(g * 16, 16)] = acc
            return carry2

        lax.fori_loop(0, C // 16, group, 0)
        return carry

    lax.fori_loop(0, K, chunk, 0)
    pltpu.sync_copy(obuf, out.at[wid])


_dec_call = functools.partial(
    pl.kernel,
    out_type=jax.ShapeDtypeStruct((NW, K, C), jnp.float32),
    mesh=plsc.VectorSubcoreMesh(**_MESH),
    compiler_params=_SC_PARAMS,
    scratch_types=[
        pltpu.VMEM((K, C), jnp.int32),
        pltpu.VMEM((K, C), jnp.int32),
        pltpu.VMEM((C, DZ), jnp.float32),
        pltpu.VMEM((C, DZ), jnp.float32),
        pltpu.VMEM((16, 16), jnp.float32),
        pltpu.VMEM((K, C), jnp.float32),
        pltpu.SemaphoreType.DMA,
    ],
)(_dec_body)


_B = 512
_GRID = N_PAD // _B


def _dinv_of(degt):
    return lax.rsqrt(degt[:, 0:1] + degt[:, 1:2] + 1.0)


def _tc1_body(degt_ref, x_ref, w1_ref, o_ref):
    dinv = _dinv_of(degt_ref[...])
    h = jnp.dot(x_ref[...], w1_ref[...], preferred_element_type=jnp.float32)
    o_ref[...] = h * dinv


def _tc1(degt, x_pad, w1):
    return pl.pallas_call(
        _tc1_body,
        grid=(_GRID,),
        in_specs=[
            pl.BlockSpec((_B, NC), lambda i: (i, 0)),
            pl.BlockSpec((_B, DIN), lambda i: (i, 0)),
            pl.BlockSpec((DIN, DH), lambda i: (0, 0)),
        ],
        out_specs=pl.BlockSpec((_B, DH), lambda i: (i, 0)),
        out_shape=jax.ShapeDtypeStruct((N_PAD, DH), jnp.float32),
    )(degt, x_pad, w1)


def _tc2_body(degt_ref, p_ref, hs1_ref, b1_ref, w2_ref, o_ref):
    dinv = _dinv_of(degt_ref[...])
    agg = p_ref[0] + p_ref[1] + hs1_ref[...]
    a1 = jnp.maximum(dinv * agg + b1_ref[...], 0.0)
    h2 = jnp.dot(a1, w2_ref[...], preferred_element_type=jnp.float32)
    o_ref[...] = h2 * dinv


def _tc2(degt, parts1, hs1, b1, w2):
    return pl.pallas_call(
        _tc2_body,
        grid=(_GRID,),
        in_specs=[
            pl.BlockSpec((_B, NC), lambda i: (i, 0)),
            pl.BlockSpec((NC, _B, DH), lambda i: (0, i, 0)),
            pl.BlockSpec((_B, DH), lambda i: (i, 0)),
            pl.BlockSpec((1, DH), lambda i: (0, 0)),
            pl.BlockSpec((DH, DZ), lambda i: (0, 0)),
        ],
        out_specs=pl.BlockSpec((_B, DZ), lambda i: (i, 0)),
        out_shape=jax.ShapeDtypeStruct((N_PAD, DZ), jnp.float32),
    )(degt, parts1, hs1, b1, w2)


def _tc3_body(degt_ref, p_ref, hs2_ref, b2_ref, o_ref):
    dinv = _dinv_of(degt_ref[...])
    agg = p_ref[0] + p_ref[1] + hs2_ref[...]
    o_ref[...] = dinv * agg + b2_ref[...]


def _tc3(degt, parts2, hs2, b2):
    return pl.pallas_call(
        _tc3_body,
        grid=(_GRID,),
        in_specs=[
            pl.BlockSpec((_B, NC), lambda i: (i, 0)),
            pl.BlockSpec((NC, _B, DZ), lambda i: (0, i, 0)),
            pl.BlockSpec((_B, DZ), lambda i: (i, 0)),
            pl.BlockSpec((1, DZ), lambda i: (0, 0)),
        ],
        out_specs=pl.BlockSpec((_B, DZ), lambda i: (i, 0)),
        out_shape=jax.ShapeDtypeStruct((N_PAD, DZ), jnp.float32),
    )(degt, parts2, hs2, b2)


@jax.jit
def kernel(x, edge_index, W1, b1, W2, b2):
    src = edge_index[0]
    dst = edge_index[1]
    pad = jnp.full((E_PAD - E,), N, jnp.int32)
    srcp = jnp.concatenate([src, pad]).reshape(NW, K, C)
    dstp = jnp.concatenate([dst, pad]).reshape(NW, K, C)
    x_pad = jnp.pad(x, ((0, N_PAD - N), (0, 0)))

    deg_parts = _deg_call(dstp)
    degt = deg_parts.T

    hs1 = _tc1(degt, x_pad, W1)
    parts1 = _seg_call_h(hs1, srcp, dstp)
    hs2 = _tc2(degt, parts1, hs1, b1.reshape(1, DH), W2)
    parts2 = _seg_call_z(hs2, srcp, dstp)
    z = _tc3(degt, parts2, hs2, b2.reshape(1, DZ))

    recon = _dec_call(z, srcp, dstp).reshape(-1)[:E]
    return z[:N], recon


# speedup vs baseline: 14.6096x; 1.3857x over previous
import functools

import jax
import jax.numpy as jnp
from jax import lax
from jax.experimental import pallas as pl
from jax.experimental.pallas import tpu as pltpu
from jax.experimental.pallas import tpu_sc as plsc

N = 10000
E = 320000
DIN = 128
DH = 64
DZ = 32

NC = 2
NS = 16
NW = NC * NS
C = 128
K = 80
EW = K * C
E_PAD = NW * EW

N_PAD = 10240
NSLICE = N_PAD // NS

_MESH = dict(core_axis_name="c", subcore_axis_name="s", num_cores=NC,
             num_subcores=NS)
_SC_PARAMS = pltpu.CompilerParams(use_tc_tiling_on_sc=False,
                                  needs_layout_passes=False)


def _wids():
    cid = lax.axis_index("c")
    sid = lax.axis_index("s")
    return cid, sid, sid * NC + cid


def _deg_body(dstp, out, idx_v, ones_v, zb_v, shared):
    cid, sid, wid = _wids()
    for t in range(C // 16):
        ones_v[pl.ds(t * 16, 16)] = jnp.ones((16,), jnp.float32)
        zb_v[pl.ds(t * 16, 16)] = jnp.zeros((16,), jnp.float32)
    base = sid * NSLICE
    for t in range(NSLICE // C):
        pltpu.sync_copy(zb_v, shared.at[pl.ds(base + t * C, C)])
    pltpu.sync_copy(dstp.at[wid], idx_v)
    plsc.subcore_barrier()

    def step(j, carry):
        pltpu.sync_copy(ones_v, shared.at[idx_v.at[j]], add=True)
        return carry

    lax.fori_loop(0, K, step, 0)
    plsc.subcore_barrier()
    pltpu.sync_copy(shared.at[pl.ds(base, NSLICE)],
                    out.at[cid].at[pl.ds(base, NSLICE)])


_deg_call = functools.partial(
    pl.kernel,
    out_type=jax.ShapeDtypeStruct((NC, N_PAD), jnp.float32),
    mesh=plsc.VectorSubcoreMesh(**_MESH),
    compiler_params=_SC_PARAMS,
    scratch_types=[
        pltpu.VMEM((K, C), jnp.int32),
        pltpu.VMEM((C,), jnp.float32),
        pltpu.VMEM((C,), jnp.float32),
        pltpu.VMEM_SHARED((N_PAD,), jnp.float32),
    ],
)(_deg_body)


def _seg_body(table, srcp, dstp, out, idx_s, idx_d, rows0, rows1, shared,
              gsem0, gsem1, ssem0, ssem1, *, depth):
    cid, sid, wid = _wids()

    def zrow(r, carry):
        for t in range(depth // 16):
            rows0[r, pl.ds(t * 16, 16)] = jnp.zeros((16,), jnp.float32)
        return carry

    lax.fori_loop(0, C, zrow, 0)
    base = sid * NSLICE
    for t in range(NSLICE // C):
        pltpu.sync_copy(rows0, shared.at[pl.ds(base + t * C, C)])
    pltpu.sync_copy(srcp.at[wid], idx_s)
    pltpu.sync_copy(dstp.at[wid], idx_d)
    plsc.subcore_barrier()

    def gat(j, rows, sem):
        pltpu.async_copy(table.at[idx_s.at[j]], rows, sem)

    def gat_wait(j, rows, sem):
        pltpu.make_async_copy(table.at[idx_s.at[j]], rows, sem).wait()

    def scat(j, rows, sem):
        pltpu.async_copy(rows, shared.at[idx_d.at[j]], sem, add=True)

    def scat_wait(j, rows, sem):
        pltpu.make_async_copy(rows, shared.at[idx_d.at[j]], sem).wait()

    gat(0, rows0, gsem0)
    gat(1, rows1, gsem1)

    def step(jj, carry):
        j0 = 2 * jj
        j1 = j0 + 1
        gat_wait(j0, rows0, gsem0)
        scat(j0, rows0, ssem0)
        scat_wait(j0, rows0, ssem0)

        @pl.when(j0 + 2 < K)
        def _():
            gat(j0 + 2, rows0, gsem0)

        gat_wait(j1, rows1, gsem1)
        scat(j1, rows1, ssem1)
        scat_wait(j1, rows1, ssem1)

        @pl.when(j1 + 2 < K)
        def _():
            gat(j1 + 2, rows1, gsem1)

        return carry

    lax.fori_loop(0, K // 2, step, 0)
    plsc.subcore_barrier()
    for t in range(NSLICE // C):
        sl = pl.ds(base + t * C, C)
        pltpu.sync_copy(shared.at[sl], out.at[cid].at[sl])


def _make_seg_call(depth):
    return functools.partial(
        pl.kernel,
        out_type=jax.ShapeDtypeStruct((NC, N_PAD, depth), jnp.float32),
        mesh=plsc.VectorSubcoreMesh(**_MESH),
        compiler_params=_SC_PARAMS,
        scratch_types=[
            pltpu.VMEM((K, C), jnp.int32),
            pltpu.VMEM((K, C), jnp.int32),
            pltpu.VMEM((C, depth), jnp.float32),
            pltpu.VMEM((C, depth), jnp.float32),
            pltpu.VMEM_SHARED((N_PAD, depth), jnp.float32),
            pltpu.SemaphoreType.DMA,
            pltpu.SemaphoreType.DMA,
            pltpu.SemaphoreType.DMA,
            pltpu.SemaphoreType.DMA,
        ],
    )(functools.partial(_seg_body, depth=depth))


_seg_call_h = _make_seg_call(DH)
_seg_call_z = _make_seg_call(DZ)


def _dec_body(ztab, srcp, dstp, out, idx_s, idx_d, zs0, zd0, zs1, zd1, fb0,
              fb1, obuf, sem0, sem1):
    cid, sid, wid = _wids()
    pltpu.sync_copy(srcp.at[wid], idx_s)
    pltpu.sync_copy(dstp.at[wid], idx_d)
    iota = lax.iota(jnp.int32, 16)

    def gat(j, zs, zd, sem):
        pltpu.async_copy(ztab.at[idx_s.at[j]], zs, sem)
        pltpu.async_copy(ztab.at[idx_d.at[j]], zd, sem)

    def gat_wait(j, zs, zd, sem):
        pltpu.make_async_copy(ztab.at[idx_s.at[j]], zs, sem).wait()
        pltpu.make_async_copy(ztab.at[idx_d.at[j]], zd, sem).wait()

    def compute(j, zs, zd):
        for g in range(C // 16):
            fb = fb0 if g % 2 == 0 else fb1
            for r in range(16):
                e = g * 16 + r
                fb[r] = (zs[e, pl.ds(0, 16)] * zd[e, pl.ds(0, 16)]
                         + zs[e, pl.ds(16, 16)] * zd[e, pl.ds(16, 16)])
            acc = plsc.load_gather(fb, [iota, jnp.zeros((16,), jnp.int32)])
            for cc in range(1, 16):
                acc = acc + plsc.load_gather(
                    fb, [iota, jnp.full((16,), cc, jnp.int32)])
            obuf[j, pl.ds(g * 16, 16)] = acc

    gat(0, zs0, zd0, sem0)
    gat(1, zs1, zd1, sem1)

    def step(jj, carry):
        j0 = 2 * jj
        j1 = j0 + 1
        gat_wait(j0, zs0, zd0, sem0)
        compute(j0, zs0, zd0)

        @pl.when(j0 + 2 < K)
        def _():
            gat(j0 + 2, zs0, zd0, sem0)

        gat_wait(j1, zs1, zd1, sem1)
        compute(j1, zs1, zd1)

        @pl.when(j1 + 2 < K)
        def _():
            gat(j1 + 2, zs1, zd1, sem1)

        return carry

    lax.fori_loop(0, K // 2, step, 0)
    pltpu.sync_copy(obuf, out.at[wid])


_dec_call = functools.partial(
    pl.kernel,
    out_type=jax.ShapeDtypeStruct((NW, K, C), jnp.float32),
    mesh=plsc.VectorSubcoreMesh(**_MESH),
    compiler_params=_SC_PARAMS,
    scratch_types=[
        pltpu.VMEM((K, C), jnp.int32),
        pltpu.VMEM((K, C), jnp.int32),
        pltpu.VMEM((C, DZ), jnp.float32),
        pltpu.VMEM((C, DZ), jnp.float32),
        pltpu.VMEM((C, DZ), jnp.float32),
        pltpu.VMEM((C, DZ), jnp.float32),
        pltpu.VMEM((16, 16), jnp.float32),
        pltpu.VMEM((16, 16), jnp.float32),
        pltpu.VMEM((K, C), jnp.float32),
        pltpu.SemaphoreType.DMA,
        pltpu.SemaphoreType.DMA,
    ],
)(_dec_body)


_B = 512
_GRID = N_PAD // _B


def _dinv_of(degt):
    return lax.rsqrt(degt[:, 0:1] + degt[:, 1:2] + 1.0)


def _tc1_body(degt_ref, x_ref, w1_ref, o_ref):
    dinv = _dinv_of(degt_ref[...])
    h = jnp.dot(x_ref[...], w1_ref[...], preferred_element_type=jnp.float32)
    o_ref[...] = h * dinv


def _tc1(degt, x_pad, w1):
    return pl.pallas_call(
        _tc1_body,
        grid=(_GRID,),
        in_specs=[
            pl.BlockSpec((_B, NC), lambda i: (i, 0)),
            pl.BlockSpec((_B, DIN), lambda i: (i, 0)),
            pl.BlockSpec((DIN, DH), lambda i: (0, 0)),
        ],
        out_specs=pl.BlockSpec((_B, DH), lambda i: (i, 0)),
        out_shape=jax.ShapeDtypeStruct((N_PAD, DH), jnp.float32),
    )(degt, x_pad, w1)


def _tc2_body(degt_ref, p_ref, hs1_ref, b1_ref, w2_ref, o_ref):
    dinv = _dinv_of(degt_ref[...])
    agg = p_ref[0] + p_ref[1] + hs1_ref[...]
    a1 = jnp.maximum(dinv * agg + b1_ref[...], 0.0)
    h2 = jnp.dot(a1, w2_ref[...], preferred_element_type=jnp.float32)
    o_ref[...] = h2 * dinv


def _tc2(degt, parts1, hs1, b1, w2):
    return pl.pallas_call(
        _tc2_body,
        grid=(_GRID,),
        in_specs=[
            pl.BlockSpec((_B, NC), lambda i: (i, 0)),
            pl.BlockSpec((NC, _B, DH), lambda i: (0, i, 0)),
            pl.BlockSpec((_B, DH), lambda i: (i, 0)),
            pl.BlockSpec((1, DH), lambda i: (0, 0)),
            pl.BlockSpec((DH, DZ), lambda i: (0, 0)),
        ],
        out_specs=pl.BlockSpec((_B, DZ), lambda i: (i, 0)),
        out_shape=jax.ShapeDtypeStruct((N_PAD, DZ), jnp.float32),
    )(degt, parts1, hs1, b1, w2)


def _tc3_body(degt_ref, p_ref, hs2_ref, b2_ref, o_ref):
    dinv = _dinv_of(degt_ref[...])
    agg = p_ref[0] + p_ref[1] + hs2_ref[...]
    o_ref[...] = dinv * agg + b2_ref[...]


def _tc3(degt, parts2, hs2, b2):
    return pl.pallas_call(
        _tc3_body,
        grid=(_GRID,),
        in_specs=[
            pl.BlockSpec((_B, NC), lambda i: (i, 0)),
            pl.BlockSpec((NC, _B, DZ), lambda i: (0, i, 0)),
            pl.BlockSpec((_B, DZ), lambda i: (i, 0)),
            pl.BlockSpec((1, DZ), lambda i: (0, 0)),
        ],
        out_specs=pl.BlockSpec((_B, DZ), lambda i: (i, 0)),
        out_shape=jax.ShapeDtypeStruct((N_PAD, DZ), jnp.float32),
    )(degt, parts2, hs2, b2)


@jax.jit
def kernel(x, edge_index, W1, b1, W2, b2):
    src = edge_index[0]
    dst = edge_index[1]
    pad = jnp.full((E_PAD - E,), N, jnp.int32)
    srcp = jnp.concatenate([src, pad]).reshape(NW, K, C)
    dstp = jnp.concatenate([dst, pad]).reshape(NW, K, C)
    x_pad = jnp.pad(x, ((0, N_PAD - N), (0, 0)))

    deg_parts = _deg_call(dstp)
    degt = deg_parts.T

    hs1 = _tc1(degt, x_pad, W1)
    parts1 = _seg_call_h(hs1, srcp, dstp)
    hs2 = _tc2(degt, parts1, hs1, b1.reshape(1, DH), W2)
    parts2 = _seg_call_z(hs2, srcp, dstp)
    z = _tc3(degt, parts2, hs2, b2.reshape(1, DZ))

    recon = _dec_call(z, srcp, dstp).reshape(-1)[:E]
    return z[:N], recon
